# initial kernel scaffold (unmeasured)
import jax
import jax.numpy as jnp
from jax import lax
from jax.experimental import pallas as pl
from jax.experimental.pallas import tpu as pltpu

N_DEV = 8
COMM_DTYPE = jnp.float32


def kernel(x, w_mat):
    m_total, k_loc = x.shape
    _, n = w_mat.shape
    m_blk = m_total // N_DEV

    def body(
        x_ref,
        w_ref,
        out_ref,
        acc_ref,
        send_ref,
        recv_ref,
        amax_src,
        amax_ref,
        send_sems,
        recv_sems,
        amax_send_sems,
        amax_recv_sems,
    ):
        my = lax.axis_index("i")
        left = lax.rem(my + N_DEV - 1, N_DEV)
        right = lax.rem(my + 1, N_DEV)

        amax_ref[:, :] = jnp.zeros_like(amax_ref)

        barrier_sem = pltpu.get_barrier_semaphore()
        for nbr in (left, right):
            pl.semaphore_signal(
                barrier_sem,
                inc=1,
                device_id=(nbr,),
                device_id_type=pl.DeviceIdType.MESH,
            )
        pl.semaphore_wait(barrier_sem, 2)

        def partial_block(b):
            xb = x_ref[pl.ds(b * m_blk, m_blk), :]
            return jnp.dot(
                xb.astype(jnp.bfloat16),
                w_ref[:, :].astype(jnp.bfloat16),
                preferred_element_type=jnp.float32,
            )

        b0 = lax.rem(my + N_DEV - 1, N_DEV)
        send_ref[:, :] = partial_block(b0).astype(COMM_DTYPE)

        acc = None
        for s in range(N_DEV - 1):
            rdma = pltpu.make_async_remote_copy(
                src_ref=send_ref,
                dst_ref=recv_ref.at[s],
                send_sem=send_sems.at[s],
                recv_sem=recv_sems.at[s],
                device_id=(right,),
                device_id_type=pl.DeviceIdType.MESH,
            )
            rdma.start()
            rdma.wait()
            b_recv = lax.rem(my + 2 * N_DEV - s - 2, N_DEV)
            acc = partial_block(b_recv) + recv_ref[s].astype(jnp.float32)
            if s < N_DEV - 2:
                send_ref[:, :] = acc.astype(COMM_DTYPE)
        acc_ref[:, :] = acc

        local_amax = jnp.max(jnp.abs(acc_ref[:, :]))
        amax_src[:, :] = jnp.full_like(amax_src, local_amax)
        for o in range(1, N_DEV):
            p = lax.rem(my + o, N_DEV)
            rdma = pltpu.make_async_remote_copy(
                src_ref=amax_src,
                dst_ref=amax_ref.at[pl.ds(my, 1)],
                send_sem=amax_send_sems.at[o],
                recv_sem=amax_recv_sems.at[my],
                device_id=(p,),
                device_id_type=pl.DeviceIdType.MESH,
            )
            rdma.start()
            rdma.wait_send()
        for o in range(1, N_DEV):
            p = lax.rem(my + o, N_DEV)
            recv = pltpu.make_async_remote_copy(
                src_ref=amax_src,
                dst_ref=amax_ref.at[pl.ds(p, 1)],
                send_sem=amax_send_sems.at[o],
                recv_sem=amax_recv_sems.at[p],
                device_id=(p,),
                device_id_type=pl.DeviceIdType.MESH,
            )
            recv.wait_recv()

        amax = jnp.maximum(local_amax, jnp.max(amax_ref[:, :]))
        scale = amax / 127.0
        q = jnp.clip(jnp.round(acc_ref[:, :] / scale), -127.0, 127.0)
        out_ref[:, :] = q * scale

    return pl.pallas_call(
        body,
        out_shape=jax.ShapeDtypeStruct((m_blk, n), jnp.float32),
        in_specs=[
            pl.BlockSpec(memory_space=pltpu.VMEM),
            pl.BlockSpec(memory_space=pltpu.VMEM),
        ],
        out_specs=pl.BlockSpec(memory_space=pltpu.VMEM),
        scratch_shapes=[
            pltpu.VMEM((m_blk, n), jnp.float32),
            pltpu.VMEM((m_blk, n), COMM_DTYPE),
            pltpu.VMEM((N_DEV - 1, m_blk, n), COMM_DTYPE),
            pltpu.VMEM((1, 128), jnp.float32),
            pltpu.VMEM((N_DEV, 128), jnp.float32),
            pltpu.SemaphoreType.DMA((N_DEV - 1,)),
            pltpu.SemaphoreType.DMA((N_DEV - 1,)),
            pltpu.SemaphoreType.DMA((N_DEV,)),
            pltpu.SemaphoreType.DMA((N_DEV,)),
        ],
        compiler_params=pltpu.CompilerParams(collective_id=0),
    )(x, w_mat)


# baseline (device time: 360300 ns/iter reference)
import jax
import jax.numpy as jnp
from jax import lax
from jax.experimental import pallas as pl
from jax.experimental.pallas import tpu as pltpu

N_DEV = 8
COMM_DTYPE = jnp.float32


def kernel(x, w_mat):
    m_total, k_loc = x.shape
    _, n = w_mat.shape
    m_blk = m_total // N_DEV

    def body(
        x_ref,
        w_ref,
        out_ref,
        acc_ref,
        send_ref,
        recv_ref,
        amax_src,
        amax_ref,
        send_sems,
        recv_sems,
        amax_send_sems,
        amax_recv_sems,
    ):
        my = lax.axis_index("i")
        left = lax.rem(my + N_DEV - 1, N_DEV)
        right = lax.rem(my + 1, N_DEV)

        amax_ref[:, :] = jnp.zeros((N_DEV, 128), jnp.float32)

        barrier_sem = pltpu.get_barrier_semaphore()
        for nbr in (left, right):
            pl.semaphore_signal(
                barrier_sem,
                inc=1,
                device_id=(nbr,),
                device_id_type=pl.DeviceIdType.MESH,
            )
        pl.semaphore_wait(barrier_sem, 2)

        def partial_block(b):
            xb = x_ref[pl.ds(b * m_blk, m_blk), :]
            return jnp.dot(
                xb.astype(jnp.bfloat16),
                w_ref[:, :].astype(jnp.bfloat16),
                preferred_element_type=jnp.float32,
            )

        b0 = lax.rem(my + N_DEV - 1, N_DEV)
        send_ref[:, :] = partial_block(b0).astype(COMM_DTYPE)

        acc = None
        for s in range(N_DEV - 1):
            rdma = pltpu.make_async_remote_copy(
                src_ref=send_ref,
                dst_ref=recv_ref.at[s],
                send_sem=send_sems.at[s],
                recv_sem=recv_sems.at[s],
                device_id=(right,),
                device_id_type=pl.DeviceIdType.MESH,
            )
            rdma.start()
            rdma.wait()
            b_recv = lax.rem(my + 2 * N_DEV - s - 2, N_DEV)
            acc = partial_block(b_recv) + recv_ref[s].astype(jnp.float32)
            if s < N_DEV - 2:
                send_ref[:, :] = acc.astype(COMM_DTYPE)
        acc_ref[:, :] = acc

        local_amax = jnp.max(jnp.abs(acc_ref[:, :]))
        amax_src[:, :] = jnp.full((1, 128), local_amax, jnp.float32)
        for o in range(1, N_DEV):
            p = lax.rem(my + o, N_DEV)
            rdma = pltpu.make_async_remote_copy(
                src_ref=amax_src,
                dst_ref=amax_ref.at[pl.ds(my, 1)],
                send_sem=amax_send_sems.at[o],
                recv_sem=amax_recv_sems.at[my],
                device_id=(p,),
                device_id_type=pl.DeviceIdType.MESH,
            )
            rdma.start()
            rdma.wait_send()
        for o in range(1, N_DEV):
            p = lax.rem(my + o, N_DEV)
            recv = pltpu.make_async_remote_copy(
                src_ref=amax_src,
                dst_ref=amax_ref.at[pl.ds(p, 1)],
                send_sem=amax_send_sems.at[o],
                recv_sem=amax_recv_sems.at[p],
                device_id=(p,),
                device_id_type=pl.DeviceIdType.MESH,
            )
            recv.wait_recv()

        amax = jnp.maximum(local_amax, jnp.max(amax_ref[:, :]))
        scale = amax / 127.0
        q = jnp.clip(jnp.round(acc_ref[:, :] / scale), -127.0, 127.0)
        out_ref[:, :] = q * scale

    return pl.pallas_call(
        body,
        out_shape=jax.ShapeDtypeStruct((m_blk, n), jnp.float32),
        in_specs=[
            pl.BlockSpec(memory_space=pltpu.VMEM),
            pl.BlockSpec(memory_space=pltpu.VMEM),
        ],
        out_specs=pl.BlockSpec(memory_space=pltpu.VMEM),
        scratch_shapes=[
            pltpu.VMEM((m_blk, n), jnp.float32),
            pltpu.VMEM((m_blk, n), COMM_DTYPE),
            pltpu.VMEM((N_DEV - 1, m_blk, n), COMM_DTYPE),
            pltpu.VMEM((1, 128), jnp.float32),
            pltpu.VMEM((N_DEV, 128), jnp.float32),
            pltpu.SemaphoreType.DMA((N_DEV - 1,)),
            pltpu.SemaphoreType.DMA((N_DEV - 1,)),
            pltpu.SemaphoreType.DMA((N_DEV,)),
            pltpu.SemaphoreType.DMA((N_DEV,)),
        ],
        compiler_params=pltpu.CompilerParams(
            collective_id=0, vmem_limit_bytes=100 * 1024 * 1024
        ),
    )(x, w_mat)


# device time: 202847 ns/iter; 1.7762x vs baseline; 1.7762x over previous
import jax
import jax.numpy as jnp
from jax import lax
from jax.experimental import pallas as pl
from jax.experimental.pallas import tpu as pltpu

N_DEV = 8
COMM_DTYPE = jnp.bfloat16


def kernel(x, w_mat):
    m_total, k_loc = x.shape
    _, n = w_mat.shape
    m_blk = m_total // N_DEV

    def body(
        x_ref,
        w_ref,
        out_ref,
        acc_ref,
        send_ref,
        recv_ref,
        amax_src,
        amax_ref,
        send_sems,
        recv_sems,
        amax_send_sems,
        amax_recv_sems,
    ):
        my = lax.axis_index("i")
        left = lax.rem(my + N_DEV - 1, N_DEV)
        right = lax.rem(my + 1, N_DEV)

        amax_ref[:, :] = jnp.zeros((N_DEV, 128), jnp.float32)

        barrier_sem = pltpu.get_barrier_semaphore()
        for nbr in (left, right):
            pl.semaphore_signal(
                barrier_sem,
                inc=1,
                device_id=(nbr,),
                device_id_type=pl.DeviceIdType.MESH,
            )
        pl.semaphore_wait(barrier_sem, 2)

        def partial_block(b):
            xb = x_ref[pl.ds(b * m_blk, m_blk), :]
            return jnp.dot(
                xb.astype(jnp.bfloat16),
                w_ref[:, :].astype(jnp.bfloat16),
                preferred_element_type=jnp.float32,
            )

        b0 = lax.rem(my + N_DEV - 1, N_DEV)
        send_ref[:, :] = partial_block(b0).astype(COMM_DTYPE)

        acc = None
        for s in range(N_DEV - 1):
            rdma = pltpu.make_async_remote_copy(
                src_ref=send_ref,
                dst_ref=recv_ref.at[s],
                send_sem=send_sems.at[s],
                recv_sem=recv_sems.at[s],
                device_id=(right,),
                device_id_type=pl.DeviceIdType.MESH,
            )
            rdma.start()
            rdma.wait()
            b_recv = lax.rem(my + 2 * N_DEV - s - 2, N_DEV)
            acc = partial_block(b_recv) + recv_ref[s].astype(jnp.float32)
            if s < N_DEV - 2:
                send_ref[:, :] = acc.astype(COMM_DTYPE)
        acc_ref[:, :] = acc

        local_amax = jnp.max(jnp.abs(acc_ref[:, :]))
        amax_src[:, :] = jnp.full((1, 128), local_amax, jnp.float32)
        for o in range(1, N_DEV):
            p = lax.rem(my + o, N_DEV)
            rdma = pltpu.make_async_remote_copy(
                src_ref=amax_src,
                dst_ref=amax_ref.at[pl.ds(my, 1)],
                send_sem=amax_send_sems.at[o],
                recv_sem=amax_recv_sems.at[my],
                device_id=(p,),
                device_id_type=pl.DeviceIdType.MESH,
            )
            rdma.start()
            rdma.wait_send()
        for o in range(1, N_DEV):
            p = lax.rem(my + o, N_DEV)
            recv = pltpu.make_async_remote_copy(
                src_ref=amax_src,
                dst_ref=amax_ref.at[pl.ds(p, 1)],
                send_sem=amax_send_sems.at[o],
                recv_sem=amax_recv_sems.at[p],
                device_id=(p,),
                device_id_type=pl.DeviceIdType.MESH,
            )
            recv.wait_recv()

        amax = jnp.maximum(local_amax, jnp.max(amax_ref[:, :]))
        scale = amax / 127.0
        q = jnp.clip(jnp.round(acc_ref[:, :] / scale), -127.0, 127.0)
        out_ref[:, :] = q * scale

    return pl.pallas_call(
        body,
        out_shape=jax.ShapeDtypeStruct((m_blk, n), jnp.float32),
        in_specs=[
            pl.BlockSpec(memory_space=pltpu.VMEM),
            pl.BlockSpec(memory_space=pltpu.VMEM),
        ],
        out_specs=pl.BlockSpec(memory_space=pltpu.VMEM),
        scratch_shapes=[
            pltpu.VMEM((m_blk, n), jnp.float32),
            pltpu.VMEM((m_blk, n), COMM_DTYPE),
            pltpu.VMEM((N_DEV - 1, m_blk, n), COMM_DTYPE),
            pltpu.VMEM((1, 128), jnp.float32),
            pltpu.VMEM((N_DEV, 128), jnp.float32),
            pltpu.SemaphoreType.DMA((N_DEV - 1,)),
            pltpu.SemaphoreType.DMA((N_DEV - 1,)),
            pltpu.SemaphoreType.DMA((N_DEV,)),
            pltpu.SemaphoreType.DMA((N_DEV,)),
        ],
        compiler_params=pltpu.CompilerParams(
            collective_id=0, vmem_limit_bytes=100 * 1024 * 1024
        ),
    )(x, w_mat)


# device time: 127811 ns/iter; 2.8190x vs baseline; 1.5871x over previous
import jax
import jax.numpy as jnp
from jax import lax
from jax.experimental import pallas as pl
from jax.experimental.pallas import tpu as pltpu

N_DEV = 8
COMM_DTYPE = jnp.bfloat16


def kernel(x, w_mat):
    m_total, k_loc = x.shape
    _, n = w_mat.shape
    m_blk = m_total // N_DEV
    h = n // 2

    def body(
        x_ref,
        w_ref,
        out_ref,
        acc_ref,
        send_a,
        send_b,
        recv_a,
        recv_b,
        amax_src,
        amax_ref,
        send_sems_a,
        recv_sems_a,
        send_sems_b,
        recv_sems_b,
        amax_send_sems,
        amax_recv_sems,
    ):
        my = lax.axis_index("i")
        left = lax.rem(my + N_DEV - 1, N_DEV)
        right = lax.rem(my + 1, N_DEV)

        amax_ref[:, :] = jnp.zeros((N_DEV, 128), jnp.float32)

        barrier_sem = pltpu.get_barrier_semaphore()
        for nbr in (left, right):
            pl.semaphore_signal(
                barrier_sem,
                inc=1,
                device_id=(nbr,),
                device_id_type=pl.DeviceIdType.MESH,
            )
        pl.semaphore_wait(barrier_sem, 2)

        def partial_a(b):
            xb = x_ref[pl.ds(b * m_blk, m_blk), :]
            return jnp.dot(
                xb.astype(jnp.bfloat16),
                w_ref[:, :h].astype(jnp.bfloat16),
                preferred_element_type=jnp.float32,
            )

        def partial_b(b):
            xb = x_ref[pl.ds(b * m_blk, m_blk), :]
            return jnp.dot(
                xb.astype(jnp.bfloat16),
                w_ref[:, h:].astype(jnp.bfloat16),
                preferred_element_type=jnp.float32,
            )

        def mk_rdma(src, dst, ssem, rsem, dev):
            return pltpu.make_async_remote_copy(
                src_ref=src,
                dst_ref=dst,
                send_sem=ssem,
                recv_sem=rsem,
                device_id=(dev,),
                device_id_type=pl.DeviceIdType.MESH,
            )

        send_a[0] = partial_a(lax.rem(my + N_DEV - 1, N_DEV)).astype(COMM_DTYPE)
        send_b[0] = partial_b(lax.rem(my + 1, N_DEV)).astype(COMM_DTYPE)
        rdma_a = [None] * (N_DEV - 1)
        rdma_b = [None] * (N_DEV - 1)
        rdma_a[0] = mk_rdma(
            send_a.at[0], recv_a.at[0], send_sems_a.at[0], recv_sems_a.at[0], right
        )
        rdma_b[0] = mk_rdma(
            send_b.at[0], recv_b.at[0], send_sems_b.at[0], recv_sems_b.at[0], left
        )
        rdma_a[0].start()
        rdma_b[0].start()

        for s in range(N_DEV - 1):
            ba = lax.rem(my + 2 * N_DEV - s - 2, N_DEV)
            bb = lax.rem(my + s + 2, N_DEV)
            pa = partial_a(ba)
            pb = partial_b(bb)
            rdma_a[s].wait_recv()
            acc_a = pa + recv_a[s].astype(jnp.float32)
            rdma_b[s].wait_recv()
            acc_b = pb + recv_b[s].astype(jnp.float32)
            if s < N_DEV - 2:
                slot = (s + 1) % 2
                if s >= 1:
                    rdma_a[s - 1].wait_send()
                    rdma_b[s - 1].wait_send()
                send_a[slot] = acc_a.astype(COMM_DTYPE)
                send_b[slot] = acc_b.astype(COMM_DTYPE)
                rdma_a[s + 1] = mk_rdma(
                    send_a.at[slot],
                    recv_a.at[s + 1],
                    send_sems_a.at[s + 1],
                    recv_sems_a.at[s + 1],
                    right,
                )
                rdma_b[s + 1] = mk_rdma(
                    send_b.at[slot],
                    recv_b.at[s + 1],
                    send_sems_b.at[s + 1],
                    recv_sems_b.at[s + 1],
                    left,
                )
                rdma_a[s + 1].start()
                rdma_b[s + 1].start()
            else:
                acc_ref[:, :h] = acc_a
                acc_ref[:, h:] = acc_b
        rdma_a[N_DEV - 3].wait_send()
        rdma_b[N_DEV - 3].wait_send()
        rdma_a[N_DEV - 2].wait_send()
        rdma_b[N_DEV - 2].wait_send()

        local_amax = jnp.max(jnp.abs(acc_ref[:, :]))
        amax_src[:, :] = jnp.full((1, 128), local_amax, jnp.float32)
        for o in range(1, N_DEV):
            p = lax.rem(my + o, N_DEV)
            rdma = mk_rdma(
                amax_src,
                amax_ref.at[pl.ds(my, 1)],
                amax_send_sems.at[o],
                amax_recv_sems.at[my],
                p,
            )
            rdma.start()
            rdma.wait_send()
        for o in range(1, N_DEV):
            p = lax.rem(my + o, N_DEV)
            recv = mk_rdma(
                amax_src,
                amax_ref.at[pl.ds(p, 1)],
                amax_send_sems.at[o],
                amax_recv_sems.at[p],
                p,
            )
            recv.wait_recv()

        amax = jnp.maximum(local_amax, jnp.max(amax_ref[:, :]))
        scale = amax / 127.0
        q = jnp.clip(jnp.round(acc_ref[:, :] / scale), -127.0, 127.0)
        out_ref[:, :] = q * scale

    return pl.pallas_call(
        body,
        out_shape=jax.ShapeDtypeStruct((m_blk, n), jnp.float32),
        in_specs=[
            pl.BlockSpec(memory_space=pltpu.VMEM),
            pl.BlockSpec(memory_space=pltpu.VMEM),
        ],
        out_specs=pl.BlockSpec(memory_space=pltpu.VMEM),
        scratch_shapes=[
            pltpu.VMEM((m_blk, n), jnp.float32),
            pltpu.VMEM((2, m_blk, h), COMM_DTYPE),
            pltpu.VMEM((2, m_blk, h), COMM_DTYPE),
            pltpu.VMEM((N_DEV - 1, m_blk, h), COMM_DTYPE),
            pltpu.VMEM((N_DEV - 1, m_blk, h), COMM_DTYPE),
            pltpu.VMEM((1, 128), jnp.float32),
            pltpu.VMEM((N_DEV, 128), jnp.float32),
            pltpu.SemaphoreType.DMA((N_DEV - 1,)),
            pltpu.SemaphoreType.DMA((N_DEV - 1,)),
            pltpu.SemaphoreType.DMA((N_DEV - 1,)),
            pltpu.SemaphoreType.DMA((N_DEV - 1,)),
            pltpu.SemaphoreType.DMA((N_DEV,)),
            pltpu.SemaphoreType.DMA((N_DEV,)),
        ],
        compiler_params=pltpu.CompilerParams(
            collective_id=0, vmem_limit_bytes=100 * 1024 * 1024
        ),
    )(x, w_mat)


# device time: 126705 ns/iter; 2.8436x vs baseline; 1.0087x over previous
import jax
import jax.numpy as jnp
from jax import lax
from jax.experimental import pallas as pl
from jax.experimental.pallas import tpu as pltpu

N_DEV = 8
COMM_DTYPE = jnp.bfloat16


def kernel(x, w_mat):
    m_total, k_loc = x.shape
    _, n = w_mat.shape
    m_blk = m_total // N_DEV
    h = n // 2

    def body(
        x_ref,
        w_ref,
        out_ref,
        acc_ref,
        send_a,
        send_b,
        recv_a,
        recv_b,
        amax_src,
        amax_ref,
        send_sems_a,
        recv_sems_a,
        send_sems_b,
        recv_sems_b,
        amax_send_sems,
        amax_recv_sems,
    ):
        my = lax.axis_index("i")

        def perm(p):
            return jnp.where(p < 4, p, 11 - p)

        pos = perm(my)
        left = perm(lax.rem(pos + N_DEV - 1, N_DEV))
        right = perm(lax.rem(pos + 1, N_DEV))

        amax_ref[:, :] = jnp.zeros((N_DEV, 128), jnp.float32)

        barrier_sem = pltpu.get_barrier_semaphore()
        for nbr in (left, right):
            pl.semaphore_signal(
                barrier_sem,
                inc=1,
                device_id=(nbr,),
                device_id_type=pl.DeviceIdType.MESH,
            )
        pl.semaphore_wait(barrier_sem, 2)

        def partial_a(b):
            xb = x_ref[pl.ds(b * m_blk, m_blk), :]
            return jnp.dot(
                xb.astype(jnp.bfloat16),
                w_ref[:, :h].astype(jnp.bfloat16),
                preferred_element_type=jnp.float32,
            )

        def partial_b(b):
            xb = x_ref[pl.ds(b * m_blk, m_blk), :]
            return jnp.dot(
                xb.astype(jnp.bfloat16),
                w_ref[:, h:].astype(jnp.bfloat16),
                preferred_element_type=jnp.float32,
            )

        def mk_rdma(src, dst, ssem, rsem, dev):
            return pltpu.make_async_remote_copy(
                src_ref=src,
                dst_ref=dst,
                send_sem=ssem,
                recv_sem=rsem,
                device_id=(dev,),
                device_id_type=pl.DeviceIdType.MESH,
            )

        send_a[0] = partial_a(left).astype(COMM_DTYPE)
        send_b[0] = partial_b(right).astype(COMM_DTYPE)
        rdma_a = [None] * (N_DEV - 1)
        rdma_b = [None] * (N_DEV - 1)
        rdma_a[0] = mk_rdma(
            send_a.at[0], recv_a.at[0], send_sems_a.at[0], recv_sems_a.at[0], right
        )
        rdma_b[0] = mk_rdma(
            send_b.at[0], recv_b.at[0], send_sems_b.at[0], recv_sems_b.at[0], left
        )
        rdma_a[0].start()
        rdma_b[0].start()

        for s in range(N_DEV - 1):
            ba = perm(lax.rem(pos + 2 * N_DEV - s - 2, N_DEV))
            bb = perm(lax.rem(pos + s + 2, N_DEV))
            pa = partial_a(ba)
            pb = partial_b(bb)
            rdma_a[s].wait_recv()
            acc_a = pa + recv_a[s].astype(jnp.float32)
            rdma_b[s].wait_recv()
            acc_b = pb + recv_b[s].astype(jnp.float32)
            if s < N_DEV - 2:
                slot = (s + 1) % 2
                if s >= 1:
                    rdma_a[s - 1].wait_send()
                    rdma_b[s - 1].wait_send()
                send_a[slot] = acc_a.astype(COMM_DTYPE)
                send_b[slot] = acc_b.astype(COMM_DTYPE)
                rdma_a[s + 1] = mk_rdma(
                    send_a.at[slot],
                    recv_a.at[s + 1],
                    send_sems_a.at[s + 1],
                    recv_sems_a.at[s + 1],
                    right,
                )
                rdma_b[s + 1] = mk_rdma(
                    send_b.at[slot],
                    recv_b.at[s + 1],
                    send_sems_b.at[s + 1],
                    recv_sems_b.at[s + 1],
                    left,
                )
                rdma_a[s + 1].start()
                rdma_b[s + 1].start()
            else:
                acc_ref[:, :h] = acc_a
                acc_ref[:, h:] = acc_b
        rdma_a[N_DEV - 3].wait_send()
        rdma_b[N_DEV - 3].wait_send()
        rdma_a[N_DEV - 2].wait_send()
        rdma_b[N_DEV - 2].wait_send()

        local_amax = jnp.max(jnp.abs(acc_ref[:, :]))
        amax_src[:, :] = jnp.full((1, 128), local_amax, jnp.float32)
        for o in range(1, N_DEV):
            p = lax.rem(my + o, N_DEV)
            rdma = mk_rdma(
                amax_src,
                amax_ref.at[pl.ds(my, 1)],
                amax_send_sems.at[o],
                amax_recv_sems.at[my],
                p,
            )
            rdma.start()
            rdma.wait_send()
        for o in range(1, N_DEV):
            p = lax.rem(my + o, N_DEV)
            recv = mk_rdma(
                amax_src,
                amax_ref.at[pl.ds(p, 1)],
                amax_send_sems.at[o],
                amax_recv_sems.at[p],
                p,
            )
            recv.wait_recv()

        amax = jnp.maximum(local_amax, jnp.max(amax_ref[:, :]))
        scale = amax / 127.0
        q = jnp.clip(jnp.round(acc_ref[:, :] / scale), -127.0, 127.0)
        out_ref[:, :] = q * scale

    return pl.pallas_call(
        body,
        out_shape=jax.ShapeDtypeStruct((m_blk, n), jnp.float32),
        in_specs=[
            pl.BlockSpec(memory_space=pltpu.VMEM),
            pl.BlockSpec(memory_space=pltpu.VMEM),
        ],
        out_specs=pl.BlockSpec(memory_space=pltpu.VMEM),
        scratch_shapes=[
            pltpu.VMEM((m_blk, n), jnp.float32),
            pltpu.VMEM((2, m_blk, h), COMM_DTYPE),
            pltpu.VMEM((2, m_blk, h), COMM_DTYPE),
            pltpu.VMEM((N_DEV - 1, m_blk, h), COMM_DTYPE),
            pltpu.VMEM((N_DEV - 1, m_blk, h), COMM_DTYPE),
            pltpu.VMEM((1, 128), jnp.float32),
            pltpu.VMEM((N_DEV, 128), jnp.float32),
            pltpu.SemaphoreType.DMA((N_DEV - 1,)),
            pltpu.SemaphoreType.DMA((N_DEV - 1,)),
            pltpu.SemaphoreType.DMA((N_DEV - 1,)),
            pltpu.SemaphoreType.DMA((N_DEV - 1,)),
            pltpu.SemaphoreType.DMA((N_DEV,)),
            pltpu.SemaphoreType.DMA((N_DEV,)),
        ],
        compiler_params=pltpu.CompilerParams(
            collective_id=0, vmem_limit_bytes=100 * 1024 * 1024
        ),
    )(x, w_mat)


# device time: 94515 ns/iter; 3.8121x vs baseline; 1.3406x over previous
import jax
import jax.numpy as jnp
from jax import lax
from jax.experimental import pallas as pl
from jax.experimental.pallas import tpu as pltpu

N_DEV = 8
COMM_DTYPE = jnp.bfloat16
GROUP_COLS = ((0, 640), (640, 1280), (1280, 2048))
DIM_MASKS = (1, 3, 4)


def kernel(x, w_mat):
    m_total, k_loc = x.shape
    _, n = w_mat.shape
    m_blk = m_total // N_DEV

    def body(x_ref, w_ref, out_ref, *rest):
        acc = rest[0:3]
        st_send0 = rest[3:6]
        st_recv0 = rest[6:9]
        st_recv1 = rest[9:12]
        st_recv2 = rest[12:15]
        (
            amax_src,
            amax_ref,
            send_sems,
            recv_sems,
            amax_send_sems,
            amax_recv_sems,
        ) = rest[15:]

        my = lax.axis_index("i")

        amax_ref[:, :] = jnp.zeros((N_DEV, 128), jnp.float32)

        barrier_sem = pltpu.get_barrier_semaphore()
        for o in range(1, N_DEV):
            pl.semaphore_signal(
                barrier_sem,
                inc=1,
                device_id=(lax.rem(my + o, N_DEV),),
                device_id_type=pl.DeviceIdType.MESH,
            )
        pl.semaphore_wait(barrier_sem, N_DEV - 1)

        def partial(o, c0, c1):
            xb = x_ref[pl.ds(o * m_blk, m_blk), :]
            return jnp.dot(
                xb.astype(jnp.bfloat16),
                w_ref[:, c0:c1].astype(jnp.bfloat16),
                preferred_element_type=jnp.float32,
            )

        def slab(ref, j, nrows=1):
            return ref.at[pl.ds(j * m_blk, nrows * m_blk), :]

        def mk(src, dst, sem_idx, partner):
            return pltpu.make_async_remote_copy(
                src_ref=src,
                dst_ref=dst,
                send_sem=send_sems.at[sem_idx],
                recv_sem=recv_sems.at[sem_idx],
                device_id=(partner,),
                device_id_type=pl.DeviceIdType.MESH,
            )

        CUR = [DIM_MASKS[g % 3] for g in range(3)]
        MA = [DIM_MASKS[(g + 1) % 3] for g in range(3)]
        MB = [DIM_MASKS[(g + 2) % 3] for g in range(3)]
        FS = [[0, MA[g], MB[g], MA[g] ^ MB[g]] for g in range(3)]

        def add_bf16(dst_ref, dj, recv_ref, rj):
            d = pl.ds(dj * m_blk, m_blk)
            r = pl.ds(rj * m_blk, m_blk)
            dst_ref[d, :] = (
                dst_ref[d, :].astype(jnp.float32)
                + recv_ref[r, :].astype(jnp.float32)
            ).astype(COMM_DTYPE)

        rd0 = []
        for g in range(3):
            c0, c1 = GROUP_COLS[g]
            for j, f in enumerate(FS[g]):
                st_send0[g][pl.ds(j * m_blk, m_blk), :] = partial(
                    my ^ (CUR[g] ^ f), c0, c1
                ).astype(COMM_DTYPE)
            r = mk(st_send0[g], st_recv0[g], 4 * g, my ^ CUR[g])
            r.start()
            rd0.append(r)
        for g in range(3):
            c0, c1 = GROUP_COLS[g]
            for j, f in enumerate(FS[g]):
                acc[g][pl.ds(j * m_blk, m_blk), :] = partial(
                    my ^ f, c0, c1
                ).astype(COMM_DTYPE)

        rd1 = []
        for g in range(3):
            rd0[g].wait_recv()
            for j in range(4):
                add_bf16(acc[g], j, st_recv0[g], j)
            p1 = my ^ MA[g]
            ra = mk(slab(acc[g], 1), slab(st_recv1[g], 0), 4 * g + 1, p1)
            rb = mk(slab(acc[g], 3), slab(st_recv1[g], 1), 4 * g + 2, p1)
            ra.start()
            rb.start()
            rd1.append((ra, rb))

        rd2 = []
        for g in range(3):
            rd1[g][0].wait_recv()
            rd1[g][1].wait_recv()
            add_bf16(acc[g], 0, st_recv1[g], 0)
            add_bf16(acc[g], 2, st_recv1[g], 1)
            r = mk(slab(acc[g], 2), st_recv2[g], 4 * g + 3, my ^ MB[g])
            r.start()
            rd2.append(r)

        for g in range(3):
            c0, c1 = GROUP_COLS[g]
            rd2[g].wait_recv()
            out_ref[:, c0:c1] = acc[g][pl.ds(0, m_blk), :].astype(
                jnp.float32
            ) + st_recv2[g][:, :].astype(jnp.float32)

        for g in range(3):
            rd0[g].wait_send()
            rd1[g][0].wait_send()
            rd1[g][1].wait_send()
            rd2[g].wait_send()

        local_amax = jnp.max(jnp.abs(out_ref[:, :]))
        amax_src[:, :] = jnp.full((1, 128), local_amax, jnp.float32)
        for o in range(1, N_DEV):
            p = lax.rem(my + o, N_DEV)
            rdma = pltpu.make_async_remote_copy(
                src_ref=amax_src,
                dst_ref=amax_ref.at[pl.ds(my, 1)],
                send_sem=amax_send_sems.at[o],
                recv_sem=amax_recv_sems.at[my],
                device_id=(p,),
                device_id_type=pl.DeviceIdType.MESH,
            )
            rdma.start()
            rdma.wait_send()
        for o in range(1, N_DEV):
            p = lax.rem(my + o, N_DEV)
            recv = pltpu.make_async_remote_copy(
                src_ref=amax_src,
                dst_ref=amax_ref.at[pl.ds(p, 1)],
                send_sem=amax_send_sems.at[o],
                recv_sem=amax_recv_sems.at[p],
                device_id=(p,),
                device_id_type=pl.DeviceIdType.MESH,
            )
            recv.wait_recv()

        amax = jnp.maximum(local_amax, jnp.max(amax_ref[:, :]))
        scale = amax / 127.0
        q = jnp.clip(jnp.round(out_ref[:, :] / scale), -127.0, 127.0)
        out_ref[:, :] = q * scale

    widths = [c1 - c0 for c0, c1 in GROUP_COLS]
    stage = lambda rows: [
        pltpu.VMEM((rows * m_blk, w), COMM_DTYPE) for w in widths
    ]

    return pl.pallas_call(
        body,
        out_shape=jax.ShapeDtypeStruct((m_blk, n), jnp.float32),
        in_specs=[
            pl.BlockSpec(memory_space=pltpu.VMEM),
            pl.BlockSpec(memory_space=pltpu.VMEM),
        ],
        out_specs=pl.BlockSpec(memory_space=pltpu.VMEM),
        scratch_shapes=[
            *stage(4),
            *stage(4),
            *stage(4),
            *stage(2),
            *stage(1),
            pltpu.VMEM((1, 128), jnp.float32),
            pltpu.VMEM((N_DEV, 128), jnp.float32),
            pltpu.SemaphoreType.DMA((12,)),
            pltpu.SemaphoreType.DMA((12,)),
            pltpu.SemaphoreType.DMA((N_DEV,)),
            pltpu.SemaphoreType.DMA((N_DEV,)),
        ],
        compiler_params=pltpu.CompilerParams(
            collective_id=0, vmem_limit_bytes=60 * 1024 * 1024
        ),
    )(x, w_mat)


# device time: 91845 ns/iter; 3.9229x vs baseline; 1.0291x over previous
import jax
import jax.numpy as jnp
from jax import lax
from jax.experimental import pallas as pl
from jax.experimental.pallas import tpu as pltpu

N_DEV = 8
COMM_DTYPE = jnp.bfloat16
GROUP_COLS = ((0, 640), (640, 1280), (1280, 2048))
DIM_MASKS = (1, 3, 4)


def kernel(x, w_mat):
    m_total, k_loc = x.shape
    _, n = w_mat.shape
    m_blk = m_total // N_DEV

    def body(x_ref, w_ref, out_ref, *rest):
        acc = rest[0:3]
        st_send0 = rest[3:6]
        st_recv0 = rest[6:9]
        st_recv1 = rest[9:12]
        st_recv2 = rest[12:15]
        (
            x_bf,
            w_bf,
            amax_src,
            amax_ref,
            send_sems,
            recv_sems,
            amax_send_sems,
            amax_recv_sems,
        ) = rest[15:]

        my = lax.axis_index("i")

        amax_ref[:, :] = jnp.zeros((N_DEV, 128), jnp.float32)

        barrier_sem = pltpu.get_barrier_semaphore()
        for o in range(1, N_DEV):
            pl.semaphore_signal(
                barrier_sem,
                inc=1,
                device_id=(lax.rem(my + o, N_DEV),),
                device_id_type=pl.DeviceIdType.MESH,
            )
        pl.semaphore_wait(barrier_sem, N_DEV - 1)

        x_bf[:, :] = x_ref[:, :].astype(jnp.bfloat16)
        w_bf[:, :] = w_ref[:, :].astype(jnp.bfloat16)

        def partial(o, c0, c1):
            return jnp.dot(
                x_bf[pl.ds(o * m_blk, m_blk), :],
                w_bf[:, c0:c1],
                preferred_element_type=jnp.float32,
            )

        def slab(ref, j, nrows=1):
            return ref.at[pl.ds(j * m_blk, nrows * m_blk), :]

        def mk(src, dst, sem_idx, partner):
            return pltpu.make_async_remote_copy(
                src_ref=src,
                dst_ref=dst,
                send_sem=send_sems.at[sem_idx],
                recv_sem=recv_sems.at[sem_idx],
                device_id=(partner,),
                device_id_type=pl.DeviceIdType.MESH,
            )

        CUR = [DIM_MASKS[g % 3] for g in range(3)]
        MA = [DIM_MASKS[(g + 1) % 3] for g in range(3)]
        MB = [DIM_MASKS[(g + 2) % 3] for g in range(3)]
        FS = [[0, MA[g], MB[g], MA[g] ^ MB[g]] for g in range(3)]

        def add_bf16(dst_ref, dj, recv_ref, rj):
            d = pl.ds(dj * m_blk, m_blk)
            r = pl.ds(rj * m_blk, m_blk)
            dst_ref[d, :] = (
                dst_ref[d, :].astype(jnp.float32)
                + recv_ref[r, :].astype(jnp.float32)
            ).astype(COMM_DTYPE)

        rd0 = []
        for g in range(3):
            c0, c1 = GROUP_COLS[g]
            for j, f in enumerate(FS[g]):
                st_send0[g][pl.ds(j * m_blk, m_blk), :] = partial(
                    my ^ (CUR[g] ^ f), c0, c1
                ).astype(COMM_DTYPE)
            r = mk(st_send0[g], st_recv0[g], 4 * g, my ^ CUR[g])
            r.start()
            rd0.append(r)
        for g in range(3):
            c0, c1 = GROUP_COLS[g]
            for j, f in enumerate(FS[g]):
                acc[g][pl.ds(j * m_blk, m_blk), :] = partial(
                    my ^ f, c0, c1
                ).astype(COMM_DTYPE)

        rd1 = []
        for g in range(3):
            rd0[g].wait_recv()
            add_bf16(acc[g], 1, st_recv0[g], 1)
            add_bf16(acc[g], 3, st_recv0[g], 3)
            p1 = my ^ MA[g]
            ra = mk(slab(acc[g], 1), slab(st_recv1[g], 0), 4 * g + 1, p1)
            rb = mk(slab(acc[g], 3), slab(st_recv1[g], 1), 4 * g + 2, p1)
            ra.start()
            rb.start()
            rd1.append((ra, rb))
        for g in range(3):
            add_bf16(acc[g], 0, st_recv0[g], 0)
            add_bf16(acc[g], 2, st_recv0[g], 2)

        rd2 = []
        for g in range(3):
            rd1[g][1].wait_recv()
            add_bf16(acc[g], 2, st_recv1[g], 1)
            r = mk(slab(acc[g], 2), st_recv2[g], 4 * g + 3, my ^ MB[g])
            r.start()
            rd2.append(r)
        for g in range(3):
            rd1[g][0].wait_recv()
            add_bf16(acc[g], 0, st_recv1[g], 0)

        for g in range(3):
            c0, c1 = GROUP_COLS[g]
            rd2[g].wait_recv()
            out_ref[:, c0:c1] = acc[g][pl.ds(0, m_blk), :].astype(
                jnp.float32
            ) + st_recv2[g][:, :].astype(jnp.float32)

        for g in range(3):
            rd0[g].wait_send()
            rd1[g][0].wait_send()
            rd1[g][1].wait_send()
            rd2[g].wait_send()

        local_amax = jnp.max(jnp.abs(out_ref[:, :]))
        amax_src[:, :] = jnp.full((1, 128), local_amax, jnp.float32)
        amax_rd = []
        for o in range(1, N_DEV):
            p = lax.rem(my + o, N_DEV)
            rdma = pltpu.make_async_remote_copy(
                src_ref=amax_src,
                dst_ref=amax_ref.at[pl.ds(my, 1)],
                send_sem=amax_send_sems.at[o],
                recv_sem=amax_recv_sems.at[my],
                device_id=(p,),
                device_id_type=pl.DeviceIdType.MESH,
            )
            rdma.start()
            amax_rd.append(rdma)
        for o in range(1, N_DEV):
            p = lax.rem(my + o, N_DEV)
            recv = pltpu.make_async_remote_copy(
                src_ref=amax_src,
                dst_ref=amax_ref.at[pl.ds(p, 1)],
                send_sem=amax_send_sems.at[o],
                recv_sem=amax_recv_sems.at[p],
                device_id=(p,),
                device_id_type=pl.DeviceIdType.MESH,
            )
            recv.wait_recv()
        for rdma in amax_rd:
            rdma.wait_send()

        amax = jnp.maximum(local_amax, jnp.max(amax_ref[:, :]))
        scale = amax / 127.0
        q = jnp.clip(jnp.round(out_ref[:, :] / scale), -127.0, 127.0)
        out_ref[:, :] = q * scale

    widths = [c1 - c0 for c0, c1 in GROUP_COLS]
    stage = lambda rows: [
        pltpu.VMEM((rows * m_blk, w), COMM_DTYPE) for w in widths
    ]

    return pl.pallas_call(
        body,
        out_shape=jax.ShapeDtypeStruct((m_blk, n), jnp.float32),
        in_specs=[
            pl.BlockSpec(memory_space=pltpu.VMEM),
            pl.BlockSpec(memory_space=pltpu.VMEM),
        ],
        out_specs=pl.BlockSpec(memory_space=pltpu.VMEM),
        scratch_shapes=[
            *stage(4),
            *stage(4),
            *stage(4),
            *stage(2),
            *stage(1),
            pltpu.VMEM((m_total, k_loc), jnp.bfloat16),
            pltpu.VMEM((k_loc, n), jnp.bfloat16),
            pltpu.VMEM((1, 128), jnp.float32),
            pltpu.VMEM((N_DEV, 128), jnp.float32),
            pltpu.SemaphoreType.DMA((12,)),
            pltpu.SemaphoreType.DMA((12,)),
            pltpu.SemaphoreType.DMA((N_DEV,)),
            pltpu.SemaphoreType.DMA((N_DEV,)),
        ],
        compiler_params=pltpu.CompilerParams(
            collective_id=0, vmem_limit_bytes=60 * 1024 * 1024
        ),
    )(x, w_mat)
